# baseline (speedup 1.0000x reference)
import functools

import jax
import jax.numpy as jnp
from jax import lax
from jax.experimental import pallas as pl
from jax.experimental.pallas import tpu as pltpu
from jax.experimental.pallas import tpu_sc as plsc

NC = 2
NS = 16
L = 16
NW = NC * NS
BLK = 128
CHUNK = 5120
NBUF = 4


NSPLIT = 4


def kernel(batch, W_a, W_b):
    B, H = batch.shape
    total = B * H
    nA, D = W_a.shape
    part = total // NSPLIT
    hpart = H // NSPLIT
    per_w = part // NW
    n_chunks = per_w // CHUNK
    nvr = CHUNK // L
    maxrow = CHUNK // BLK + 2
    nb_max = CHUNK // BLK + 1

    mesh = plsc.VectorSubcoreMesh(core_axis_name="c", subcore_axis_name="s")

    @functools.partial(
        pl.kernel,
        out_type=jax.ShapeDtypeStruct((part, D), jnp.float32),
        mesh=mesh,
        compiler_params=pltpu.CompilerParams(use_tc_tiling_on_sc=False,
                                             needs_layout_passes=False),
        scratch_types=[
            pltpu.VMEM((CHUNK,), jnp.int32),
            pltpu.VMEM((maxrow, BLK), jnp.int32),
            pltpu.VMEM((maxrow, BLK), jnp.int32),
            pltpu.VMEM((maxrow, BLK), jnp.int32),
            pltpu.VMEM((maxrow, BLK), jnp.int32),
            pltpu.VMEM((NBUF, BLK, 64), jnp.float32),
        ] + [pltpu.SemaphoreType.DMA] * (2 * NBUF),
    )
    def sc_kernel(idx_hbm, wa_hbm, wb_hbm, out_hbm,
                  idx_v, idxA, posA, idxB, posB, rows, *sems):
        gsem = sems[:NBUF]
        ssem = sems[NBUF:]
        wid = lax.axis_index("s") * NC + lax.axis_index("c")
        wbase = wid * per_w
        iota = lax.iota(jnp.int32, L)

        def lane0(vec):
            return jnp.zeros((L,), jnp.int32) + jnp.sum(
                jnp.where(iota == 0, vec, 0))

        def chunk_body(c, _):
            off = wbase + c * CHUNK
            pltpu.sync_copy(idx_hbm.at[pl.ds(off, CHUNK)], idx_v)

            def compact(v, carry):
                offA, offB = carry
                ids = idx_v[pl.ds(v * L, L)]
                maskB = ids >= nA
                maskA = ids < nA
                mA = maskA.astype(jnp.int32)
                inclA = plsc.cumsum(mA)
                exclA = inclA - mA
                cA = jnp.max(inclA)
                posv = off + v * L + iota
                tgtA = offA + exclA
                plsc.store_scatter(idxA, [tgtA >> 7, tgtA & 127], ids,
                                   mask=maskA)
                plsc.store_scatter(posA, [tgtA >> 7, tgtA & 127], posv,
                                   mask=maskA)
                tgtB = offB + (iota - exclA)
                plsc.store_scatter(idxB, [tgtB >> 7, tgtB & 127], ids - nA,
                                   mask=maskB)
                plsc.store_scatter(posB, [tgtB >> 7, tgtB & 127], posv,
                                   mask=maskB)
                return offA + cA, offB + (L - cA)

            cntA, cntB = lax.fori_loop(
                0, nvr, compact, (jnp.int32(0), jnp.int32(0)))

            ka = (cntA + BLK - 1) >> 7
            kb = (cntB + BLK - 1) >> 7
            padidA = lane0(idxA[0, pl.ds(0, L)])
            padposA = lane0(posA[0, pl.ds(0, L)])
            padidB = lane0(idxB[0, pl.ds(0, L)])
            padposB = lane0(posB[0, pl.ds(0, L)])
            for t in range(BLK // L):
                tA = cntA + t * L + iota
                mA_ = tA < ka * BLK
                plsc.store_scatter(idxA, [tA >> 7, tA & 127], padidA,
                                   mask=mA_)
                plsc.store_scatter(posA, [tA >> 7, tA & 127], padposA,
                                   mask=mA_)
                tB = cntB + t * L + iota
                mB_ = tB < kb * BLK
                plsc.store_scatter(idxB, [tB >> 7, tB & 127], padidB,
                                   mask=mB_)
                plsc.store_scatter(posB, [tB >> 7, tB & 127], padposB,
                                   mask=mB_)

            jtot = ka + kb

            def gather_blk(b, slot):
                @pl.when(b < ka)
                def _():
                    pltpu.async_copy(wa_hbm.at[idxA.at[b]], rows.at[slot],
                                     gsem[slot])

                @pl.when(b >= ka)
                def _():
                    pltpu.async_copy(wb_hbm.at[idxB.at[b - ka]],
                                     rows.at[slot], gsem[slot])

            def scatter_blk(b, slot):
                @pl.when(b < ka)
                def _():
                    pltpu.async_copy(rows.at[slot], out_hbm.at[posA.at[b]],
                                     ssem[slot])

                @pl.when(b >= ka)
                def _():
                    pltpu.async_copy(rows.at[slot], out_hbm.at[posB.at[b - ka]],
                                     ssem[slot])

            def wait_gather(slot):
                pltpu.make_async_copy(wa_hbm.at[idxA.at[0]], rows.at[slot],
                                      gsem[slot]).wait()

            def wait_scatter(slot):
                pltpu.make_async_copy(rows.at[slot],
                                      out_hbm.at[posA.at[0]],
                                      ssem[slot]).wait()

            for j in range(nb_max + NBUF - 1):
                g = j
                s = j - (NBUF - 1)
                if g < nb_max:
                    slot = g % NBUF

                    @pl.when(g < jtot)
                    def _(g=g, slot=slot):
                        if g >= NBUF:
                            wait_scatter(slot)
                        gather_blk(g, slot)

                if s >= 0:
                    slot = s % NBUF

                    @pl.when(s < jtot)
                    def _(s=s, slot=slot):
                        wait_gather(slot)
                        scatter_blk(s, slot)

            for slot in range(NBUF):
                wait_scatter(slot)
            return 0

        lax.fori_loop(0, n_chunks, chunk_body, 0)

    idx3 = batch.astype(jnp.int32)
    parts = [
        sc_kernel(idx3[:, k * hpart:(k + 1) * hpart].reshape(part), W_a, W_b)
        .reshape(B, hpart, D)
        for k in range(NSPLIT)
    ]
    return jnp.concatenate(parts, axis=1)


# speedup vs baseline: 2.5563x; 2.5563x over previous
import functools

import jax
import jax.numpy as jnp
from jax import lax
from jax.experimental import pallas as pl
from jax.experimental.pallas import tpu as pltpu
from jax.experimental.pallas import tpu_sc as plsc

NC = 2
NS = 16
L = 16
NW = NC * NS
BLK = 128
CHUNK = 5120
NBUF = 4


def kernel(batch, W_a, W_b):
    B, H = batch.shape
    total = B * H
    nA, D = W_a.shape
    per_w = total // NW
    n_chunks = per_w // CHUNK
    nvr = CHUNK // L
    maxrow = CHUNK // BLK + 2
    nb_max = CHUNK // BLK + 1

    idx_flat = batch.reshape(total).astype(jnp.int32)

    mesh = plsc.VectorSubcoreMesh(core_axis_name="c", subcore_axis_name="s")

    @functools.partial(
        pl.kernel,
        out_type=jax.ShapeDtypeStruct((total, D), jnp.float32),
        mesh=mesh,
        compiler_params=pltpu.CompilerParams(use_tc_tiling_on_sc=False,
                                             needs_layout_passes=False),
        scratch_types=[
            pltpu.VMEM((CHUNK,), jnp.int32),
            pltpu.VMEM((maxrow, BLK), jnp.int32),
            pltpu.VMEM((maxrow, BLK), jnp.int32),
            pltpu.VMEM((maxrow, BLK), jnp.int32),
            pltpu.VMEM((maxrow, BLK), jnp.int32),
            pltpu.VMEM((NBUF, BLK, 64), jnp.float32),
        ] + [pltpu.SemaphoreType.DMA] * (2 * NBUF),
    )
    def sc_kernel(idx_hbm, wa_hbm, wb_hbm, out_hbm,
                  idx_v, idxA, posA, idxB, posB, rows, *sems):
        gsem = sems[:NBUF]
        ssem = sems[NBUF:]
        wid = lax.axis_index("s") * NC + lax.axis_index("c")
        wbase = wid * per_w
        iota = lax.iota(jnp.int32, L)

        def lane0(vec):
            return jnp.zeros((L,), jnp.int32) + jnp.sum(
                jnp.where(iota == 0, vec, 0))

        def chunk_body(c, _):
            off = wbase + c * CHUNK

            def wait_gather(slot):
                pltpu.make_async_copy(wa_hbm.at[idxA.at[0]], rows.at[slot],
                                      gsem[slot]).wait()

            def wait_scatter(slot):
                pltpu.make_async_copy(rows.at[slot],
                                      out_hbm.at[posA.at[0]],
                                      ssem[slot]).wait()

            pltpu.sync_copy(idx_hbm.at[pl.ds(off, CHUNK)], idx_v)

            def compact(v, carry):
                offA, offB = carry
                ids = idx_v[pl.ds(v * L, L)]
                maskB = ids >= nA
                maskA = ids < nA
                mA = maskA.astype(jnp.int32)
                inclA = plsc.cumsum(mA)
                exclA = inclA - mA
                cA = jnp.max(inclA)
                posv = off + v * L + iota
                tgtA = offA + exclA
                plsc.store_scatter(idxA, [tgtA >> 7, tgtA & 127], ids,
                                   mask=maskA)
                plsc.store_scatter(posA, [tgtA >> 7, tgtA & 127], posv,
                                   mask=maskA)
                tgtB = offB + (iota - exclA)
                plsc.store_scatter(idxB, [tgtB >> 7, tgtB & 127], ids - nA,
                                   mask=maskB)
                plsc.store_scatter(posB, [tgtB >> 7, tgtB & 127], posv,
                                   mask=maskB)
                return offA + cA, offB + (L - cA)

            cntA, cntB = lax.fori_loop(
                0, nvr, compact, (jnp.int32(0), jnp.int32(0)))

            @pl.when(c > 0)
            def _():
                for slot in range(NBUF):
                    wait_scatter(slot)

            ka = (cntA + BLK - 1) >> 7
            kb = (cntB + BLK - 1) >> 7
            padidA = lane0(idxA[0, pl.ds(0, L)])
            padposA = lane0(posA[0, pl.ds(0, L)])
            padidB = lane0(idxB[0, pl.ds(0, L)])
            padposB = lane0(posB[0, pl.ds(0, L)])
            for t in range(BLK // L):
                tA = cntA + t * L + iota
                mA_ = tA < ka * BLK
                plsc.store_scatter(idxA, [tA >> 7, tA & 127], padidA,
                                   mask=mA_)
                plsc.store_scatter(posA, [tA >> 7, tA & 127], padposA,
                                   mask=mA_)
                tB = cntB + t * L + iota
                mB_ = tB < kb * BLK
                plsc.store_scatter(idxB, [tB >> 7, tB & 127], padidB,
                                   mask=mB_)
                plsc.store_scatter(posB, [tB >> 7, tB & 127], padposB,
                                   mask=mB_)

            jtot = ka + kb

            def gather_blk(b, slot):
                @pl.when(b < ka)
                def _():
                    pltpu.async_copy(wa_hbm.at[idxA.at[b]], rows.at[slot],
                                     gsem[slot])

                @pl.when(b >= ka)
                def _():
                    pltpu.async_copy(wb_hbm.at[idxB.at[b - ka]],
                                     rows.at[slot], gsem[slot])

            def scatter_blk(b, slot):
                @pl.when(b < ka)
                def _():
                    pltpu.async_copy(rows.at[slot], out_hbm.at[posA.at[b]],
                                     ssem[slot])

                @pl.when(b >= ka)
                def _():
                    pltpu.async_copy(rows.at[slot], out_hbm.at[posB.at[b - ka]],
                                     ssem[slot])

            for j in range(nb_max + NBUF - 1):
                g = j
                s = j - (NBUF - 1)
                if g < nb_max:
                    slot = g % NBUF

                    @pl.when(g < jtot)
                    def _(g=g, slot=slot):
                        if g >= NBUF:
                            wait_scatter(slot)
                        gather_blk(g, slot)

                if s >= 0:
                    slot = s % NBUF

                    @pl.when(s < jtot)
                    def _(s=s, slot=slot):
                        wait_gather(slot)
                        scatter_blk(s, slot)

            return 0

        lax.fori_loop(0, n_chunks, chunk_body, 0)
        for slot in range(NBUF):
            pltpu.make_async_copy(rows.at[slot], out_hbm.at[posA.at[0]],
                                  sems[NBUF + slot]).wait()

    out = sc_kernel(idx_flat, W_a, W_b)
    return out.reshape(B, H, D)


# speedup vs baseline: 2.6010x; 1.0175x over previous
import functools

import jax
import jax.numpy as jnp
from jax import lax
from jax.experimental import pallas as pl
from jax.experimental.pallas import tpu as pltpu
from jax.experimental.pallas import tpu_sc as plsc

NC = 2
NS = 16
L = 16
NW = NC * NS
BLK = 128
CHUNK = 6400
NBUF = 4


def kernel(batch, W_a, W_b):
    B, H = batch.shape
    total = B * H
    nA, D = W_a.shape
    per_w = total // NW
    n_chunks = per_w // CHUNK
    nvr = CHUNK // L
    maxrow = CHUNK // BLK + 2
    nb_max = CHUNK // BLK + 1

    idx_flat = batch.reshape(total).astype(jnp.int32)

    mesh = plsc.VectorSubcoreMesh(core_axis_name="c", subcore_axis_name="s")

    @functools.partial(
        pl.kernel,
        out_type=jax.ShapeDtypeStruct((total, D), jnp.float32),
        mesh=mesh,
        compiler_params=pltpu.CompilerParams(use_tc_tiling_on_sc=False,
                                             needs_layout_passes=False),
        scratch_types=[
            pltpu.VMEM((CHUNK,), jnp.int32),
            pltpu.VMEM((maxrow, BLK), jnp.int32),
            pltpu.VMEM((maxrow, BLK), jnp.int32),
            pltpu.VMEM((maxrow, BLK), jnp.int32),
            pltpu.VMEM((maxrow, BLK), jnp.int32),
            pltpu.VMEM((NBUF, BLK, 64), jnp.float32),
        ] + [pltpu.SemaphoreType.DMA] * (2 * NBUF),
    )
    def sc_kernel(idx_hbm, wa_hbm, wb_hbm, out_hbm,
                  idx_v, idxA, posA, idxB, posB, rows, *sems):
        gsem = sems[:NBUF]
        ssem = sems[NBUF:]
        wid = lax.axis_index("s") * NC + lax.axis_index("c")
        wbase = wid * per_w
        iota = lax.iota(jnp.int32, L)

        def lane0(vec):
            return jnp.zeros((L,), jnp.int32) + jnp.sum(
                jnp.where(iota == 0, vec, 0))

        def chunk_body(c, _):
            off = wbase + c * CHUNK

            def wait_gather(slot):
                pltpu.make_async_copy(wa_hbm.at[idxA.at[0]], rows.at[slot],
                                      gsem[slot]).wait()

            def wait_scatter(slot):
                pltpu.make_async_copy(rows.at[slot],
                                      out_hbm.at[posA.at[0]],
                                      ssem[slot]).wait()

            pltpu.sync_copy(idx_hbm.at[pl.ds(off, CHUNK)], idx_v)

            def compact(v, carry):
                offA, offB = carry
                ids = idx_v[pl.ds(v * L, L)]
                maskB = ids >= nA
                maskA = ids < nA
                mA = maskA.astype(jnp.int32)
                inclA = plsc.cumsum(mA)
                exclA = inclA - mA
                cA = jnp.max(inclA)
                posv = off + v * L + iota
                tgtA = offA + exclA
                plsc.store_scatter(idxA, [tgtA >> 7, tgtA & 127], ids,
                                   mask=maskA)
                plsc.store_scatter(posA, [tgtA >> 7, tgtA & 127], posv,
                                   mask=maskA)
                tgtB = offB + (iota - exclA)
                plsc.store_scatter(idxB, [tgtB >> 7, tgtB & 127], ids - nA,
                                   mask=maskB)
                plsc.store_scatter(posB, [tgtB >> 7, tgtB & 127], posv,
                                   mask=maskB)
                return offA + cA, offB + (L - cA)

            cntA, cntB = lax.fori_loop(
                0, nvr, compact, (jnp.int32(0), jnp.int32(0)))

            @pl.when(c > 0)
            def _():
                for slot in range(NBUF):
                    wait_scatter(slot)

            ka = (cntA + BLK - 1) >> 7
            kb = (cntB + BLK - 1) >> 7
            padidA = lane0(idxA[0, pl.ds(0, L)])
            padposA = lane0(posA[0, pl.ds(0, L)])
            padidB = lane0(idxB[0, pl.ds(0, L)])
            padposB = lane0(posB[0, pl.ds(0, L)])
            for t in range(BLK // L):
                tA = cntA + t * L + iota
                mA_ = tA < ka * BLK
                plsc.store_scatter(idxA, [tA >> 7, tA & 127], padidA,
                                   mask=mA_)
                plsc.store_scatter(posA, [tA >> 7, tA & 127], padposA,
                                   mask=mA_)
                tB = cntB + t * L + iota
                mB_ = tB < kb * BLK
                plsc.store_scatter(idxB, [tB >> 7, tB & 127], padidB,
                                   mask=mB_)
                plsc.store_scatter(posB, [tB >> 7, tB & 127], padposB,
                                   mask=mB_)

            jtot = ka + kb

            def gather_blk(b, slot):
                @pl.when(b < ka)
                def _():
                    pltpu.async_copy(wa_hbm.at[idxA.at[b]], rows.at[slot],
                                     gsem[slot])

                @pl.when(b >= ka)
                def _():
                    pltpu.async_copy(wb_hbm.at[idxB.at[b - ka]],
                                     rows.at[slot], gsem[slot])

            def scatter_blk(b, slot):
                @pl.when(b < ka)
                def _():
                    pltpu.async_copy(rows.at[slot], out_hbm.at[posA.at[b]],
                                     ssem[slot])

                @pl.when(b >= ka)
                def _():
                    pltpu.async_copy(rows.at[slot], out_hbm.at[posB.at[b - ka]],
                                     ssem[slot])

            for j in range(nb_max + NBUF - 1):
                g = j
                s = j - (NBUF - 1)
                if g < nb_max:
                    slot = g % NBUF

                    @pl.when(g < jtot)
                    def _(g=g, slot=slot):
                        if g >= NBUF:
                            wait_scatter(slot)
                        gather_blk(g, slot)

                if s >= 0:
                    slot = s % NBUF

                    @pl.when(s < jtot)
                    def _(s=s, slot=slot):
                        wait_gather(slot)
                        scatter_blk(s, slot)

            return 0

        lax.fori_loop(0, n_chunks, chunk_body, 0)
        for slot in range(NBUF):
            pltpu.make_async_copy(rows.at[slot], out_hbm.at[posA.at[0]],
                                  sems[NBUF + slot]).wait()

    out = sc_kernel(idx_flat, W_a, W_b)
    return out.reshape(B, H, D)


# speedup vs baseline: 2.6650x; 1.0246x over previous
import functools

import jax
import jax.numpy as jnp
from jax import lax
from jax.experimental import pallas as pl
from jax.experimental.pallas import tpu as pltpu
from jax.experimental.pallas import tpu_sc as plsc

NC = 2
NS = 16
L = 16
NW = NC * NS
BLK = 128
CHUNK = 6400
NBUF = 6


def kernel(batch, W_a, W_b):
    B, H = batch.shape
    total = B * H
    nA, D = W_a.shape
    per_w = total // NW
    n_chunks = per_w // CHUNK
    nvr = CHUNK // L
    maxrow = CHUNK // BLK + 2
    nb_max = CHUNK // BLK + 1

    idx_flat = batch.reshape(total).astype(jnp.int32)

    mesh = plsc.VectorSubcoreMesh(core_axis_name="c", subcore_axis_name="s")

    @functools.partial(
        pl.kernel,
        out_type=jax.ShapeDtypeStruct((total, D), jnp.float32),
        mesh=mesh,
        compiler_params=pltpu.CompilerParams(use_tc_tiling_on_sc=False,
                                             needs_layout_passes=False),
        scratch_types=[
            pltpu.VMEM((CHUNK,), jnp.int32),
            pltpu.VMEM((maxrow, BLK), jnp.int32),
            pltpu.VMEM((maxrow, BLK), jnp.int32),
            pltpu.VMEM((maxrow, BLK), jnp.int32),
            pltpu.VMEM((maxrow, BLK), jnp.int32),
            pltpu.VMEM((NBUF, BLK, 64), jnp.float32),
        ] + [pltpu.SemaphoreType.DMA] * (2 * NBUF),
    )
    def sc_kernel(idx_hbm, wa_hbm, wb_hbm, out_hbm,
                  idx_v, idxA, posA, idxB, posB, rows, *sems):
        gsem = sems[:NBUF]
        ssem = sems[NBUF:]
        wid = lax.axis_index("s") * NC + lax.axis_index("c")
        wbase = wid * per_w
        iota = lax.iota(jnp.int32, L)

        def lane0(vec):
            return jnp.zeros((L,), jnp.int32) + jnp.sum(
                jnp.where(iota == 0, vec, 0))

        def chunk_body(c, _):
            off = wbase + c * CHUNK

            def wait_gather(slot):
                pltpu.make_async_copy(wa_hbm.at[idxA.at[0]], rows.at[slot],
                                      gsem[slot]).wait()

            def wait_scatter(slot):
                pltpu.make_async_copy(rows.at[slot],
                                      out_hbm.at[posA.at[0]],
                                      ssem[slot]).wait()

            pltpu.sync_copy(idx_hbm.at[pl.ds(off, CHUNK)], idx_v)

            def compact(v, carry):
                offA, offB = carry
                ids = idx_v[pl.ds(v * L, L)]
                maskB = ids >= nA
                maskA = ids < nA
                mA = maskA.astype(jnp.int32)
                inclA = plsc.cumsum(mA)
                exclA = inclA - mA
                cA = jnp.max(inclA)
                posv = off + v * L + iota
                tgtA = offA + exclA
                plsc.store_scatter(idxA, [tgtA >> 7, tgtA & 127], ids,
                                   mask=maskA)
                plsc.store_scatter(posA, [tgtA >> 7, tgtA & 127], posv,
                                   mask=maskA)
                tgtB = offB + (iota - exclA)
                plsc.store_scatter(idxB, [tgtB >> 7, tgtB & 127], ids - nA,
                                   mask=maskB)
                plsc.store_scatter(posB, [tgtB >> 7, tgtB & 127], posv,
                                   mask=maskB)
                return offA + cA, offB + (L - cA)

            cntA, cntB = lax.fori_loop(
                0, nvr, compact, (jnp.int32(0), jnp.int32(0)))

            @pl.when(c > 0)
            def _():
                for slot in range(NBUF):
                    wait_scatter(slot)

            ka = (cntA + BLK - 1) >> 7
            kb = (cntB + BLK - 1) >> 7
            padidA = lane0(idxA[0, pl.ds(0, L)])
            padposA = lane0(posA[0, pl.ds(0, L)])
            padidB = lane0(idxB[0, pl.ds(0, L)])
            padposB = lane0(posB[0, pl.ds(0, L)])
            for t in range(BLK // L):
                tA = cntA + t * L + iota
                mA_ = tA < ka * BLK
                plsc.store_scatter(idxA, [tA >> 7, tA & 127], padidA,
                                   mask=mA_)
                plsc.store_scatter(posA, [tA >> 7, tA & 127], padposA,
                                   mask=mA_)
                tB = cntB + t * L + iota
                mB_ = tB < kb * BLK
                plsc.store_scatter(idxB, [tB >> 7, tB & 127], padidB,
                                   mask=mB_)
                plsc.store_scatter(posB, [tB >> 7, tB & 127], padposB,
                                   mask=mB_)

            jtot = ka + kb

            def gather_blk(b, slot):
                @pl.when(b < ka)
                def _():
                    pltpu.async_copy(wa_hbm.at[idxA.at[b]], rows.at[slot],
                                     gsem[slot])

                @pl.when(b >= ka)
                def _():
                    pltpu.async_copy(wb_hbm.at[idxB.at[b - ka]],
                                     rows.at[slot], gsem[slot])

            def scatter_blk(b, slot):
                @pl.when(b < ka)
                def _():
                    pltpu.async_copy(rows.at[slot], out_hbm.at[posA.at[b]],
                                     ssem[slot])

                @pl.when(b >= ka)
                def _():
                    pltpu.async_copy(rows.at[slot], out_hbm.at[posB.at[b - ka]],
                                     ssem[slot])

            for j in range(nb_max + NBUF - 1):
                g = j
                s = j - (NBUF - 1)
                if g < nb_max:
                    slot = g % NBUF

                    @pl.when(g < jtot)
                    def _(g=g, slot=slot):
                        if g >= NBUF:
                            wait_scatter(slot)
                        gather_blk(g, slot)

                if s >= 0:
                    slot = s % NBUF

                    @pl.when(s < jtot)
                    def _(s=s, slot=slot):
                        wait_gather(slot)
                        scatter_blk(s, slot)

            return 0

        lax.fori_loop(0, n_chunks, chunk_body, 0)
        for slot in range(NBUF):
            pltpu.make_async_copy(rows.at[slot], out_hbm.at[posA.at[0]],
                                  sems[NBUF + slot]).wait()

    out = sc_kernel(idx_flat, W_a, W_b)
    return out.reshape(B, H, D)


# speedup vs baseline: 2.6960x; 1.0116x over previous
import functools

import jax
import jax.numpy as jnp
from jax import lax
from jax.experimental import pallas as pl
from jax.experimental.pallas import tpu as pltpu
from jax.experimental.pallas import tpu_sc as plsc

NC = 2
NS = 16
L = 16
NW = NC * NS
BLK = 128
CHUNK = 6400
NBUF = 8


def kernel(batch, W_a, W_b):
    B, H = batch.shape
    total = B * H
    nA, D = W_a.shape
    per_w = total // NW
    n_chunks = per_w // CHUNK
    nvr = CHUNK // L
    maxrow = CHUNK // BLK + 2
    nb_max = CHUNK // BLK + 1

    idx_flat = batch.reshape(total).astype(jnp.int32)

    mesh = plsc.VectorSubcoreMesh(core_axis_name="c", subcore_axis_name="s")

    @functools.partial(
        pl.kernel,
        out_type=jax.ShapeDtypeStruct((total, D), jnp.float32),
        mesh=mesh,
        compiler_params=pltpu.CompilerParams(use_tc_tiling_on_sc=False,
                                             needs_layout_passes=False),
        scratch_types=[
            pltpu.VMEM((CHUNK,), jnp.int32),
            pltpu.VMEM((maxrow, BLK), jnp.int32),
            pltpu.VMEM((maxrow, BLK), jnp.int32),
            pltpu.VMEM((maxrow, BLK), jnp.int32),
            pltpu.VMEM((maxrow, BLK), jnp.int32),
            pltpu.VMEM((NBUF, BLK, 64), jnp.float32),
        ] + [pltpu.SemaphoreType.DMA] * (2 * NBUF),
    )
    def sc_kernel(idx_hbm, wa_hbm, wb_hbm, out_hbm,
                  idx_v, idxA, posA, idxB, posB, rows, *sems):
        gsem = sems[:NBUF]
        ssem = sems[NBUF:]
        wid = lax.axis_index("s") * NC + lax.axis_index("c")
        wbase = wid * per_w
        iota = lax.iota(jnp.int32, L)

        def lane0(vec):
            return jnp.zeros((L,), jnp.int32) + jnp.sum(
                jnp.where(iota == 0, vec, 0))

        def chunk_body(c, _):
            off = wbase + c * CHUNK

            def wait_gather(slot):
                pltpu.make_async_copy(wa_hbm.at[idxA.at[0]], rows.at[slot],
                                      gsem[slot]).wait()

            def wait_scatter(slot):
                pltpu.make_async_copy(rows.at[slot],
                                      out_hbm.at[posA.at[0]],
                                      ssem[slot]).wait()

            pltpu.sync_copy(idx_hbm.at[pl.ds(off, CHUNK)], idx_v)

            def compact(v, carry):
                offA, offB = carry
                ids = idx_v[pl.ds(v * L, L)]
                maskB = ids >= nA
                maskA = ids < nA
                mA = maskA.astype(jnp.int32)
                inclA = plsc.cumsum(mA)
                exclA = inclA - mA
                cA = jnp.max(inclA)
                posv = off + v * L + iota
                tgtA = offA + exclA
                plsc.store_scatter(idxA, [tgtA >> 7, tgtA & 127], ids,
                                   mask=maskA)
                plsc.store_scatter(posA, [tgtA >> 7, tgtA & 127], posv,
                                   mask=maskA)
                tgtB = offB + (iota - exclA)
                plsc.store_scatter(idxB, [tgtB >> 7, tgtB & 127], ids - nA,
                                   mask=maskB)
                plsc.store_scatter(posB, [tgtB >> 7, tgtB & 127], posv,
                                   mask=maskB)
                return offA + cA, offB + (L - cA)

            cntA, cntB = lax.fori_loop(
                0, nvr, compact, (jnp.int32(0), jnp.int32(0)))

            @pl.when(c > 0)
            def _():
                for slot in range(NBUF):
                    wait_scatter(slot)

            ka = (cntA + BLK - 1) >> 7
            kb = (cntB + BLK - 1) >> 7
            padidA = lane0(idxA[0, pl.ds(0, L)])
            padposA = lane0(posA[0, pl.ds(0, L)])
            padidB = lane0(idxB[0, pl.ds(0, L)])
            padposB = lane0(posB[0, pl.ds(0, L)])
            for t in range(BLK // L):
                tA = cntA + t * L + iota
                mA_ = tA < ka * BLK
                plsc.store_scatter(idxA, [tA >> 7, tA & 127], padidA,
                                   mask=mA_)
                plsc.store_scatter(posA, [tA >> 7, tA & 127], padposA,
                                   mask=mA_)
                tB = cntB + t * L + iota
                mB_ = tB < kb * BLK
                plsc.store_scatter(idxB, [tB >> 7, tB & 127], padidB,
                                   mask=mB_)
                plsc.store_scatter(posB, [tB >> 7, tB & 127], padposB,
                                   mask=mB_)

            jtot = ka + kb

            def gather_blk(b, slot):
                @pl.when(b < ka)
                def _():
                    pltpu.async_copy(wa_hbm.at[idxA.at[b]], rows.at[slot],
                                     gsem[slot])

                @pl.when(b >= ka)
                def _():
                    pltpu.async_copy(wb_hbm.at[idxB.at[b - ka]],
                                     rows.at[slot], gsem[slot])

            def scatter_blk(b, slot):
                @pl.when(b < ka)
                def _():
                    pltpu.async_copy(rows.at[slot], out_hbm.at[posA.at[b]],
                                     ssem[slot])

                @pl.when(b >= ka)
                def _():
                    pltpu.async_copy(rows.at[slot], out_hbm.at[posB.at[b - ka]],
                                     ssem[slot])

            for j in range(nb_max + NBUF - 1):
                g = j
                s = j - (NBUF - 1)
                if g < nb_max:
                    slot = g % NBUF

                    @pl.when(g < jtot)
                    def _(g=g, slot=slot):
                        if g >= NBUF:
                            wait_scatter(slot)
                        gather_blk(g, slot)

                if s >= 0:
                    slot = s % NBUF

                    @pl.when(s < jtot)
                    def _(s=s, slot=slot):
                        wait_gather(slot)
                        scatter_blk(s, slot)

            return 0

        lax.fori_loop(0, n_chunks, chunk_body, 0)
        for slot in range(NBUF):
            pltpu.make_async_copy(rows.at[slot], out_hbm.at[posA.at[0]],
                                  sems[NBUF + slot]).wait()

    out = sc_kernel(idx_flat, W_a, W_b)
    return out.reshape(B, H, D)


# speedup vs baseline: 2.7075x; 1.0043x over previous
import functools

import jax
import jax.numpy as jnp
from jax import lax
from jax.experimental import pallas as pl
from jax.experimental.pallas import tpu as pltpu
from jax.experimental.pallas import tpu_sc as plsc

NC = 2
NS = 16
L = 16
NW = NC * NS
BLK = 128
CHUNK = 6400
NBUF = 10


def kernel(batch, W_a, W_b):
    B, H = batch.shape
    total = B * H
    nA, D = W_a.shape
    per_w = total // NW
    n_chunks = per_w // CHUNK
    nvr = CHUNK // L
    maxrow = CHUNK // BLK + 2
    nb_max = CHUNK // BLK + 1

    idx_flat = batch.reshape(total).astype(jnp.int32)

    mesh = plsc.VectorSubcoreMesh(core_axis_name="c", subcore_axis_name="s")

    @functools.partial(
        pl.kernel,
        out_type=jax.ShapeDtypeStruct((total, D), jnp.float32),
        mesh=mesh,
        compiler_params=pltpu.CompilerParams(use_tc_tiling_on_sc=False,
                                             needs_layout_passes=False),
        scratch_types=[
            pltpu.VMEM((CHUNK,), jnp.int32),
            pltpu.VMEM((maxrow, BLK), jnp.int32),
            pltpu.VMEM((maxrow, BLK), jnp.int32),
            pltpu.VMEM((maxrow, BLK), jnp.int32),
            pltpu.VMEM((maxrow, BLK), jnp.int32),
            pltpu.VMEM((NBUF, BLK, 64), jnp.float32),
        ] + [pltpu.SemaphoreType.DMA] * (2 * NBUF),
    )
    def sc_kernel(idx_hbm, wa_hbm, wb_hbm, out_hbm,
                  idx_v, idxA, posA, idxB, posB, rows, *sems):
        gsem = sems[:NBUF]
        ssem = sems[NBUF:]
        wid = lax.axis_index("s") * NC + lax.axis_index("c")
        wbase = wid * per_w
        iota = lax.iota(jnp.int32, L)

        def lane0(vec):
            return jnp.zeros((L,), jnp.int32) + jnp.sum(
                jnp.where(iota == 0, vec, 0))

        def chunk_body(c, _):
            off = wbase + c * CHUNK

            def wait_gather(slot):
                pltpu.make_async_copy(wa_hbm.at[idxA.at[0]], rows.at[slot],
                                      gsem[slot]).wait()

            def wait_scatter(slot):
                pltpu.make_async_copy(rows.at[slot],
                                      out_hbm.at[posA.at[0]],
                                      ssem[slot]).wait()

            pltpu.sync_copy(idx_hbm.at[pl.ds(off, CHUNK)], idx_v)

            def compact(v, carry):
                offA, offB = carry
                ids = idx_v[pl.ds(v * L, L)]
                maskB = ids >= nA
                maskA = ids < nA
                mA = maskA.astype(jnp.int32)
                inclA = plsc.cumsum(mA)
                exclA = inclA - mA
                cA = jnp.max(inclA)
                posv = off + v * L + iota
                tgtA = offA + exclA
                plsc.store_scatter(idxA, [tgtA >> 7, tgtA & 127], ids,
                                   mask=maskA)
                plsc.store_scatter(posA, [tgtA >> 7, tgtA & 127], posv,
                                   mask=maskA)
                tgtB = offB + (iota - exclA)
                plsc.store_scatter(idxB, [tgtB >> 7, tgtB & 127], ids - nA,
                                   mask=maskB)
                plsc.store_scatter(posB, [tgtB >> 7, tgtB & 127], posv,
                                   mask=maskB)
                return offA + cA, offB + (L - cA)

            cntA, cntB = lax.fori_loop(
                0, nvr, compact, (jnp.int32(0), jnp.int32(0)))

            @pl.when(c > 0)
            def _():
                for slot in range(NBUF):
                    wait_scatter(slot)

            ka = (cntA + BLK - 1) >> 7
            kb = (cntB + BLK - 1) >> 7
            padidA = lane0(idxA[0, pl.ds(0, L)])
            padposA = lane0(posA[0, pl.ds(0, L)])
            padidB = lane0(idxB[0, pl.ds(0, L)])
            padposB = lane0(posB[0, pl.ds(0, L)])
            for t in range(BLK // L):
                tA = cntA + t * L + iota
                mA_ = tA < ka * BLK
                plsc.store_scatter(idxA, [tA >> 7, tA & 127], padidA,
                                   mask=mA_)
                plsc.store_scatter(posA, [tA >> 7, tA & 127], padposA,
                                   mask=mA_)
                tB = cntB + t * L + iota
                mB_ = tB < kb * BLK
                plsc.store_scatter(idxB, [tB >> 7, tB & 127], padidB,
                                   mask=mB_)
                plsc.store_scatter(posB, [tB >> 7, tB & 127], padposB,
                                   mask=mB_)

            jtot = ka + kb

            def gather_blk(b, slot):
                @pl.when(b < ka)
                def _():
                    pltpu.async_copy(wa_hbm.at[idxA.at[b]], rows.at[slot],
                                     gsem[slot])

                @pl.when(b >= ka)
                def _():
                    pltpu.async_copy(wb_hbm.at[idxB.at[b - ka]],
                                     rows.at[slot], gsem[slot])

            def scatter_blk(b, slot):
                @pl.when(b < ka)
                def _():
                    pltpu.async_copy(rows.at[slot], out_hbm.at[posA.at[b]],
                                     ssem[slot])

                @pl.when(b >= ka)
                def _():
                    pltpu.async_copy(rows.at[slot], out_hbm.at[posB.at[b - ka]],
                                     ssem[slot])

            for j in range(nb_max + NBUF - 1):
                g = j
                s = j - (NBUF - 1)
                if g < nb_max:
                    slot = g % NBUF

                    @pl.when(g < jtot)
                    def _(g=g, slot=slot):
                        if g >= NBUF:
                            wait_scatter(slot)
                        gather_blk(g, slot)

                if s >= 0:
                    slot = s % NBUF

                    @pl.when(s < jtot)
                    def _(s=s, slot=slot):
                        wait_gather(slot)
                        scatter_blk(s, slot)

            return 0

        lax.fori_loop(0, n_chunks, chunk_body, 0)
        for slot in range(NBUF):
            pltpu.make_async_copy(rows.at[slot], out_hbm.at[posA.at[0]],
                                  sems[NBUF + slot]).wait()

    out = sc_kernel(idx_flat, W_a, W_b)
    return out.reshape(B, H, D)
